# Initial kernel scaffold; baseline (speedup 1.0000x reference)
import jax
import jax.numpy as jnp
from jax.experimental import pallas as pl


def kernel(x, emb):
    raise NotImplementedError("write your pallas kernel here")


# speedup vs baseline: 1.6664x; 1.6664x over previous
import jax
import jax.numpy as jnp
from jax.experimental import pallas as pl


_SEQ_BLOCK = 1024


def _add_body(x_ref, emb_ref, out_ref):
    out_ref[...] = x_ref[...] + emb_ref[...][None]


def kernel(x, emb):
    batch, seq, dim = x.shape
    sb = _SEQ_BLOCK
    grid = (seq // sb, batch)
    return pl.pallas_call(
        _add_body,
        grid=grid,
        in_specs=[
            pl.BlockSpec((1, sb, dim), lambda i, j: (j, i, 0)),
            pl.BlockSpec((sb, dim), lambda i, j: (i, 0)),
        ],
        out_specs=pl.BlockSpec((1, sb, dim), lambda i, j: (j, i, 0)),
        out_shape=jax.ShapeDtypeStruct(x.shape, x.dtype),
    )(x, emb[:seq])
